# Initial kernel scaffold; baseline (speedup 1.0000x reference)
import jax
import jax.numpy as jnp
from jax.experimental import pallas as pl


def kernel(f_prediction, g0_prediction, adj_row, adj_col, adj_val, W1, b1, W2, b2, W3, b3):
    raise NotImplementedError("write your pallas kernel here")


# speedup vs baseline: 2.6728x; 2.6728x over previous
import functools

import jax
import jax.numpy as jnp
from jax import lax
from jax.experimental import pallas as pl
from jax.experimental.pallas import tpu as pltpu
from jax.experimental.pallas import tpu_sc as plsc

N = 50000
E = 800000
NCLASS = 64
TOPK = 16
DEGREE = 9

NW = 32
R = 1568
NP = NW * R
BLK = 512
GRID = NP // BLK
C = 128
EP = E + C


def _mlp_body(g0_ref, w1_ref, b1_ref, w2_ref, b2_ref, w3_ref, b3_ref, out_ref):
    x = g0_ref[...]
    m = jnp.max(x, axis=1, keepdims=True)
    e = jnp.exp(x - m)
    p = e / jnp.sum(e, axis=1, keepdims=True)
    iota = lax.broadcasted_iota(jnp.int32, p.shape, 1)
    cols = []
    cur = p
    for _ in range(TOPK):
        mx = jnp.max(cur, axis=1, keepdims=True)
        cols.append(mx)
        first = jnp.min(jnp.where(cur == mx, iota, NCLASS), axis=1, keepdims=True)
        cur = jnp.where(iota == first, -jnp.inf, cur)
    ranked = jnp.concatenate(cols, axis=1)
    hp = jax.lax.Precision.HIGHEST
    h = jnp.tanh(jnp.dot(ranked, w1_ref[...], precision=hp,
                         preferred_element_type=jnp.float32) + b1_ref[...])
    h = jnp.tanh(jnp.dot(h, w2_ref[...], precision=hp,
                         preferred_element_type=jnp.float32) + b2_ref[...])
    out_ref[...] = jnp.dot(h, w3_ref[...], precision=hp,
                           preferred_element_type=jnp.float32) + b3_ref[...]


def _mlp_call(g0_pad, W1, b1, W2, b2, W3p, b3p):
    full = lambda s: pl.BlockSpec(s, lambda i: (0, 0))
    return pl.pallas_call(
        _mlp_body,
        grid=(GRID,),
        in_specs=[
            pl.BlockSpec((BLK, NCLASS), lambda i: (i, 0)),
            full(W1.shape), full(b1.shape), full(W2.shape),
            full(b2.shape), full(W3p.shape), full(b3p.shape),
        ],
        out_specs=pl.BlockSpec((BLK, TOPK), lambda i: (i, 0)),
        out_shape=jax.ShapeDtypeStruct((NP, TOPK), jnp.float32),
    )(g0_pad, W1, b1, W2, b2, W3p, b3p)


def _sc_hop_body(fp_hbm, col_hbm, val_hbm, row_hbm, bounds_hbm, out_hbm,
                 acc, colv, valv, rowv, rowsv, bv, sem):
    cid = lax.axis_index("c")
    sid = lax.axis_index("s")
    wid = sid * 2 + cid
    pltpu.sync_copy(bounds_hbm, bv)
    bpair = bv[pl.ds(wid * 16, 16)]
    start = bpair[0]
    end = bpair[1]
    row_base = wid * R

    def zero(i, _):
        acc[pl.ds(i * 16, 16)] = jnp.zeros((16,), jnp.float32)
        return 0
    lax.fori_loop(0, R * 4, zero, 0)

    base = start - lax.rem(start, 8)
    nch = lax.div(end - base + (C - 1), C)
    lanes = lax.iota(jnp.int32, 16)

    def chunk(k, _):
        off = pl.multiple_of(base + k * C, 8)
        pltpu.sync_copy(col_hbm.at[pl.ds(off, C)], colv)
        pltpu.sync_copy(val_hbm.at[pl.ds(off, C)], valv)
        pltpu.sync_copy(row_hbm.at[pl.ds(off, C)], rowv)
        pltpu.async_copy(fp_hbm.at[colv], rowsv, sem).wait()
        for g in range(C // 16):
            eidx = off + g * 16 + lanes
            ok = (eidx >= start) & (eidx < end)
            rv = jnp.clip(rowv[pl.ds(g * 16, 16)] - row_base, 0, R - 1)
            vv = jnp.where(ok, valv[pl.ds(g * 16, 16)], 0.0)
            for i in range(16):
                r = rv[i]
                v = vv[i]
                for j in range(4):
                    plsc.addupdate(acc.at[pl.ds(r * 64 + j * 16, 16)],
                                   v * rowsv[g * 16 + i, pl.ds(j * 16, 16)])
        return 0
    lax.fori_loop(0, nch, chunk, 0)
    pltpu.sync_copy(acc, out_hbm.at[pl.ds(pl.multiple_of(row_base * 64, 8),
                                          R * 64)])


@functools.lru_cache(maxsize=None)
def _get_sc_hop():
    return functools.partial(
        pl.kernel,
        out_type=jax.ShapeDtypeStruct((NP * 64,), jnp.float32),
        mesh=plsc.VectorSubcoreMesh(core_axis_name="c", subcore_axis_name="s",
                                    num_cores=2, num_subcores=16),
        scratch_types=[
            pltpu.VMEM((R * 64,), jnp.float32),
            pltpu.VMEM((C,), jnp.int32),
            pltpu.VMEM((C,), jnp.float32),
            pltpu.VMEM((C,), jnp.int32),
            pltpu.VMEM((C, 64), jnp.float32),
            pltpu.VMEM((NW * 16,), jnp.int32),
            pltpu.SemaphoreType.DMA,
        ],
        compiler_params=pltpu.CompilerParams(use_tc_tiling_on_sc=False),
    )(_sc_hop_body)


def _combine_body(w_ref, *refs):
    fp_refs = refs[:DEGREE]
    out_ref = refs[DEGREE]
    w = w_ref[...]
    acc = w[:, 0:1] * fp_refs[0][...]
    for h in range(1, DEGREE):
        acc = acc + w[:, h:h + 1] * fp_refs[h][...]
    m = jnp.max(acc, axis=1, keepdims=True)
    s = acc - m
    out_ref[...] = s - jnp.log(jnp.sum(jnp.exp(s), axis=1, keepdims=True))


def _combine_call(weight_pad, fps):
    return pl.pallas_call(
        _combine_body,
        grid=(GRID,),
        in_specs=[pl.BlockSpec((BLK, TOPK), lambda i: (i, 0))]
        + [pl.BlockSpec((BLK, NCLASS), lambda i: (i, 0))] * DEGREE,
        out_specs=pl.BlockSpec((BLK, NCLASS), lambda i: (i, 0)),
        out_shape=jax.ShapeDtypeStruct((NP, NCLASS), jnp.float32),
    )(weight_pad, *fps)


def kernel(f_prediction, g0_prediction, adj_row, adj_col, adj_val,
           W1, b1, W2, b2, W3, b3):
    f_pad = jnp.pad(f_prediction, ((0, NP - N), (0, 0)))
    g0_pad = jnp.pad(g0_prediction, ((0, NP - N), (0, 0)))
    W3p = jnp.pad(W3, ((0, 0), (0, TOPK - DEGREE)))
    b1r = b1.reshape(1, -1)
    b2r = b2.reshape(1, -1)
    b3r = jnp.pad(b3, (0, TOPK - DEGREE)).reshape(1, -1)

    weight_pad = _mlp_call(g0_pad, W1, b1r, W2, b2r, W3p, b3r)

    row32 = adj_row.astype(jnp.int32)
    bounds = jnp.searchsorted(row32, jnp.arange(NW + 1, dtype=jnp.int32) * R,
                              side="left").astype(jnp.int32)
    bounds_arr = jnp.zeros((NW, 16), jnp.int32)
    bounds_arr = bounds_arr.at[:, 0].set(bounds[:NW])
    bounds_arr = bounds_arr.at[:, 1].set(bounds[1:]).reshape(NW * 16)
    colp = jnp.pad(adj_col.astype(jnp.int32), (0, EP - E))
    valp = jnp.pad(adj_val, (0, EP - E))
    rowp = jnp.pad(row32, (0, EP - E))

    sc_hop = _get_sc_hop()
    fps = [f_pad]
    cur = f_pad
    for _ in range(DEGREE - 1):
        cur = sc_hop(cur, colp, valp, rowp, bounds_arr).reshape(NP, NCLASS)
        fps.append(cur)

    logp_pad = _combine_call(weight_pad, fps)
    return logp_pad[:N], weight_pad[:N, :DEGREE]


# speedup vs baseline: 10.3145x; 3.8591x over previous
import functools

import jax
import jax.numpy as jnp
from jax import lax
from jax.experimental import pallas as pl
from jax.experimental.pallas import tpu as pltpu
from jax.experimental.pallas import tpu_sc as plsc

N = 50000
E = 800000
NCLASS = 64
TOPK = 16
DEGREE = 9

NW = 32
R = 1568
NP = NW * R
BLK = 512
GRID = NP // BLK
C = 128
NCH = E // C + 6


def _mlp_body(g0_ref, w1_ref, b1_ref, w2_ref, b2_ref, w3_ref, b3_ref, out_ref):
    x = g0_ref[...]
    m = jnp.max(x, axis=1, keepdims=True)
    e = jnp.exp(x - m)
    p = e / jnp.sum(e, axis=1, keepdims=True)
    iota = lax.broadcasted_iota(jnp.int32, p.shape, 1)
    cols = []
    cur = p
    for _ in range(TOPK):
        mx = jnp.max(cur, axis=1, keepdims=True)
        cols.append(mx)
        first = jnp.min(jnp.where(cur == mx, iota, NCLASS), axis=1, keepdims=True)
        cur = jnp.where(iota == first, -jnp.inf, cur)
    ranked = jnp.concatenate(cols, axis=1)
    hp = jax.lax.Precision.HIGHEST
    h = jnp.tanh(jnp.dot(ranked, w1_ref[...], precision=hp,
                         preferred_element_type=jnp.float32) + b1_ref[...])
    h = jnp.tanh(jnp.dot(h, w2_ref[...], precision=hp,
                         preferred_element_type=jnp.float32) + b2_ref[...])
    out_ref[...] = jnp.dot(h, w3_ref[...], precision=hp,
                           preferred_element_type=jnp.float32) + b3_ref[...]


def _mlp_call(g0_pad, W1, b1, W2, b2, W3p, b3p):
    full = lambda s: pl.BlockSpec(s, lambda i: (0, 0))
    return pl.pallas_call(
        _mlp_body,
        grid=(GRID,),
        in_specs=[
            pl.BlockSpec((BLK, NCLASS), lambda i: (i, 0)),
            full(W1.shape), full(b1.shape), full(W2.shape),
            full(b2.shape), full(W3p.shape), full(b3p.shape),
        ],
        out_specs=pl.BlockSpec((BLK, TOPK), lambda i: (i, 0)),
        out_shape=jax.ShapeDtypeStruct((NP, TOPK), jnp.float32),
    )(g0_pad, W1, b1, W2, b2, W3p, b3p)


def _sc_hop_body(fp_hbm, adj_hbm, bounds_hbm, out_hbm,
                 acc, adjb, rowsb, bv, sem_a, sem_g):
    cid = lax.axis_index("c")
    sid = lax.axis_index("s")
    wid = sid * 2 + cid
    pltpu.sync_copy(bounds_hbm, bv)
    bpair = bv[pl.ds(wid * 16, 16)]
    start = bpair[0]
    end = bpair[1]
    row_base = wid * R

    def zero(i, _):
        acc[pl.ds(i * 16, 16)] = jnp.zeros((16,), jnp.float32)
        return 0
    lax.fori_loop(0, R * 4, zero, 0)

    k0 = lax.div(start, C)
    nch = lax.div(end + (C - 1), C) - k0
    lanes = lax.iota(jnp.int32, 16)
    splat_idx = [jnp.full((16,), i, jnp.int32) for i in range(16)]
    off_j = [j * 16 + lanes for j in range(4)]

    pltpu.async_copy(adj_hbm.at[k0], adjb.at[0], sem_a.at[0]).wait()
    pltpu.async_copy(fp_hbm.at[adjb.at[0, 0]], rowsb.at[0], sem_g.at[0])
    pltpu.async_copy(adj_hbm.at[k0 + 1], adjb.at[1], sem_a.at[1])

    def chunk(t, _):
        a = lax.rem(t, 3)
        b = lax.rem(t, 2)
        a1 = lax.rem(t + 1, 3)
        b1 = lax.rem(t + 1, 2)
        a2 = lax.rem(t + 2, 3)
        pltpu.make_async_copy(adj_hbm.at[k0 + t + 1], adjb.at[a1],
                              sem_a.at[a1]).wait()
        pltpu.async_copy(fp_hbm.at[adjb.at[a1, 0]], rowsb.at[b1],
                         sem_g.at[b1])
        pltpu.async_copy(adj_hbm.at[k0 + t + 2], adjb.at[a2], sem_a.at[a2])
        pltpu.make_async_copy(fp_hbm.at[adjb.at[a, 0]], rowsb.at[b],
                              sem_g.at[b]).wait()
        off = (k0 + t) * C
        for g in range(C // 16):
            eidx = off + g * 16 + lanes
            ok = (eidx >= start) & (eidx < end)
            rv64 = jnp.clip(adjb[a, 2, pl.ds(g * 16, 16)] - row_base,
                            0, R - 1) * 64
            vv = jnp.where(ok, plsc.bitcast(adjb[a, 1, pl.ds(g * 16, 16)],
                                            jnp.float32), 0.0)
            for q in range(4):
                e4 = [q * 4 + u for u in range(4)]
                idx = [rv64.at[splat_idx[i]].get(mode="promise_in_bounds")
                       for i in e4]
                vs = [vv.at[splat_idx[i]].get(mode="promise_in_bounds")
                      for i in e4]
                xs = [[vs[u] * rowsb[b, g * 16 + e4[u], pl.ds(j * 16, 16)]
                       for j in range(4)] for u in range(4)]
                for u in range(4):
                    for j in range(4):
                        plsc.addupdate_scatter(acc, [idx[u] + off_j[j]],
                                               xs[u][j])
        return 0
    lax.fori_loop(0, nch, chunk, 0)

    bn = lax.rem(nch, 2)
    an = lax.rem(nch, 3)
    an1 = lax.rem(nch + 1, 3)
    pltpu.make_async_copy(fp_hbm.at[adjb.at[an, 0]], rowsb.at[bn],
                          sem_g.at[bn]).wait()
    pltpu.make_async_copy(adj_hbm.at[0], adjb.at[an1], sem_a.at[an1]).wait()
    pltpu.sync_copy(acc, out_hbm.at[pl.ds(pl.multiple_of(row_base * 64, 8),
                                          R * 64)])


@functools.lru_cache(maxsize=None)
def _get_sc_hop():
    return functools.partial(
        pl.kernel,
        out_type=jax.ShapeDtypeStruct((NP * 64,), jnp.float32),
        mesh=plsc.VectorSubcoreMesh(core_axis_name="c", subcore_axis_name="s",
                                    num_cores=2, num_subcores=16),
        scratch_types=[
            pltpu.VMEM((R * 64,), jnp.float32),
            pltpu.VMEM((3, 3, C), jnp.int32),
            pltpu.VMEM((2, C, 64), jnp.float32),
            pltpu.VMEM((NW * 16,), jnp.int32),
            pltpu.SemaphoreType.DMA((3,)),
            pltpu.SemaphoreType.DMA((2,)),
        ],
        compiler_params=pltpu.CompilerParams(use_tc_tiling_on_sc=False,
                                             needs_layout_passes=False),
    )(_sc_hop_body)


def _combine_body(w_ref, *refs):
    fp_refs = refs[:DEGREE]
    out_ref = refs[DEGREE]
    w = w_ref[...]
    acc = w[:, 0:1] * fp_refs[0][...]
    for h in range(1, DEGREE):
        acc = acc + w[:, h:h + 1] * fp_refs[h][...]
    m = jnp.max(acc, axis=1, keepdims=True)
    s = acc - m
    out_ref[...] = s - jnp.log(jnp.sum(jnp.exp(s), axis=1, keepdims=True))


def _combine_call(weight_pad, fps):
    return pl.pallas_call(
        _combine_body,
        grid=(GRID,),
        in_specs=[pl.BlockSpec((BLK, TOPK), lambda i: (i, 0))]
        + [pl.BlockSpec((BLK, NCLASS), lambda i: (i, 0))] * DEGREE,
        out_specs=pl.BlockSpec((BLK, NCLASS), lambda i: (i, 0)),
        out_shape=jax.ShapeDtypeStruct((NP, NCLASS), jnp.float32),
    )(weight_pad, *fps)


def kernel(f_prediction, g0_prediction, adj_row, adj_col, adj_val,
           W1, b1, W2, b2, W3, b3):
    f_pad = jnp.pad(f_prediction, ((0, NP - N), (0, 0)))
    g0_pad = jnp.pad(g0_prediction, ((0, NP - N), (0, 0)))
    W3p = jnp.pad(W3, ((0, 0), (0, TOPK - DEGREE)))
    b1r = b1.reshape(1, -1)
    b2r = b2.reshape(1, -1)
    b3r = jnp.pad(b3, (0, TOPK - DEGREE)).reshape(1, -1)

    weight_pad = _mlp_call(g0_pad, W1, b1r, W2, b2r, W3p, b3r)

    row32 = adj_row.astype(jnp.int32)
    bounds = jnp.searchsorted(row32, jnp.arange(NW + 1, dtype=jnp.int32) * R,
                              side="left").astype(jnp.int32)
    bounds_arr = jnp.zeros((NW, 16), jnp.int32)
    bounds_arr = bounds_arr.at[:, 0].set(bounds[:NW])
    bounds_arr = bounds_arr.at[:, 1].set(bounds[1:]).reshape(NW * 16)
    pad_e = NCH * C - E
    colp = jnp.pad(adj_col.astype(jnp.int32), (0, pad_e)).reshape(NCH, 1, C)
    valp = jnp.pad(lax.bitcast_convert_type(adj_val, jnp.int32),
                   (0, pad_e)).reshape(NCH, 1, C)
    rowp = jnp.pad(row32, (0, pad_e)).reshape(NCH, 1, C)
    adj_packed = jnp.concatenate([colp, valp, rowp], axis=1)

    sc_hop = _get_sc_hop()
    fps = [f_pad]
    cur = f_pad
    for _ in range(DEGREE - 1):
        cur = sc_hop(cur, adj_packed, bounds_arr).reshape(NP, NCLASS)
        fps.append(cur)

    logp_pad = _combine_call(weight_pad, fps)
    return logp_pad[:N], weight_pad[:N, :DEGREE]


# speedup vs baseline: 12.9447x; 1.2550x over previous
import functools

import jax
import jax.numpy as jnp
from jax import lax
from jax.experimental import pallas as pl
from jax.experimental.pallas import tpu as pltpu
from jax.experimental.pallas import tpu_sc as plsc

N = 50000
E = 800000
NCLASS = 64
TOPK = 16
DEGREE = 9

NW = 32
R = 1568
NP = NW * R
BLK = 512
GRID = NP // BLK
C = 128
NCH = E // C + 6


def _mlp_body(g0_ref, w1_ref, b1_ref, w2_ref, b2_ref, w3_ref, b3_ref, out_ref):
    x = g0_ref[...]
    m = jnp.max(x, axis=1, keepdims=True)
    e = jnp.exp(x - m)
    p = e / jnp.sum(e, axis=1, keepdims=True)
    iota = lax.broadcasted_iota(jnp.int32, p.shape, 1)
    cur = p + iota.astype(jnp.float32) * (2.0 ** -20)
    cols = []
    for _ in range(TOPK):
        mx = jnp.max(cur, axis=1, keepdims=True)
        cols.append(mx)
        cur = jnp.where(cur == mx, -jnp.inf, cur)
    ranked = jnp.concatenate(cols, axis=1)
    hp = jax.lax.Precision.HIGHEST
    h = jnp.tanh(jnp.dot(ranked, w1_ref[...], precision=hp,
                         preferred_element_type=jnp.float32) + b1_ref[...])
    h = jnp.tanh(jnp.dot(h, w2_ref[...], precision=hp,
                         preferred_element_type=jnp.float32) + b2_ref[...])
    out_ref[...] = jnp.dot(h, w3_ref[...], precision=hp,
                           preferred_element_type=jnp.float32) + b3_ref[...]


def _mlp_call(g0_pad, W1, b1, W2, b2, W3p, b3p):
    full = lambda s: pl.BlockSpec(s, lambda i: (0, 0))
    return pl.pallas_call(
        _mlp_body,
        grid=(GRID,),
        in_specs=[
            pl.BlockSpec((BLK, NCLASS), lambda i: (i, 0)),
            full(W1.shape), full(b1.shape), full(W2.shape),
            full(b2.shape), full(W3p.shape), full(b3p.shape),
        ],
        out_specs=pl.BlockSpec((BLK, TOPK), lambda i: (i, 0)),
        out_shape=jax.ShapeDtypeStruct((NP, TOPK), jnp.float32),
    )(g0_pad, W1, b1, W2, b2, W3p, b3p)


def _sc_hop_body(fp_hbm, adj_hbm, bounds_hbm, out_hbm,
                 acc, adjb, rowsb, bv, sem_a, sem_g):
    cid = lax.axis_index("c")
    sid = lax.axis_index("s")
    wid = sid * 2 + cid
    pltpu.sync_copy(bounds_hbm, bv)
    bpair = bv[pl.ds(wid * 16, 16)]
    start = bpair[0]
    end = bpair[1]
    row_base = wid * R

    z16 = jnp.zeros((16,), jnp.float32)

    def zero(i, _):
        for j in range(4):
            acc[i, pl.ds(j * 16, 16)] = z16
        return 0
    lax.fori_loop(0, R, zero, 0)

    k0 = lax.div(start, C)
    nch = lax.div(end + (C - 1), C) - k0
    lanes = lax.iota(jnp.int32, 16)
    splat_idx = [jnp.full((16,), i, jnp.int32) for i in range(16)]
    off_j = [j * 16 + lanes for j in range(4)]

    pltpu.async_copy(adj_hbm.at[k0], adjb.at[0], sem_a.at[0]).wait()
    pltpu.async_copy(fp_hbm.at[adjb.at[0, 0]], rowsb.at[0], sem_g.at[0])
    pltpu.async_copy(adj_hbm.at[k0 + 1], adjb.at[1], sem_a.at[1])

    def chunk(t, _):
        a = lax.rem(t, 3)
        b = lax.rem(t, 2)
        a1 = lax.rem(t + 1, 3)
        b1 = lax.rem(t + 1, 2)
        a2 = lax.rem(t + 2, 3)
        pltpu.make_async_copy(adj_hbm.at[k0 + t + 1], adjb.at[a1],
                              sem_a.at[a1]).wait()
        pltpu.async_copy(fp_hbm.at[adjb.at[a1, 0]], rowsb.at[b1],
                         sem_g.at[b1])
        pltpu.async_copy(adj_hbm.at[k0 + t + 2], adjb.at[a2], sem_a.at[a2])
        pltpu.make_async_copy(fp_hbm.at[adjb.at[a, 0]], rowsb.at[b],
                              sem_g.at[b]).wait()
        off = (k0 + t) * C
        for g in range(C // 16):
            eidx = off + g * 16 + lanes
            ok = (eidx >= start) & (eidx < end)
            rv = jnp.clip(adjb[a, 2, pl.ds(g * 16, 16)] - row_base, 0, R - 1)
            vv = jnp.where(ok, plsc.bitcast(adjb[a, 1, pl.ds(g * 16, 16)],
                                            jnp.float32), 0.0)
            for q in range(4):
                e4 = [q * 4 + u for u in range(4)]
                idx = [rv.at[splat_idx[i]].get(mode="promise_in_bounds")
                       for i in e4]
                vs = [vv.at[splat_idx[i]].get(mode="promise_in_bounds")
                      for i in e4]
                xs = [[vs[u] * rowsb[b, g * 16 + e4[u], pl.ds(j * 16, 16)]
                       for j in range(4)] for u in range(4)]
                for u in range(4):
                    for j in range(4):
                        plsc.addupdate_scatter(acc, [idx[u], off_j[j]],
                                               xs[u][j])
        return 0
    lax.fori_loop(0, nch, chunk, 0)

    bn = lax.rem(nch, 2)
    an = lax.rem(nch, 3)
    an1 = lax.rem(nch + 1, 3)
    pltpu.make_async_copy(fp_hbm.at[adjb.at[an, 0]], rowsb.at[bn],
                          sem_g.at[bn]).wait()
    pltpu.make_async_copy(adj_hbm.at[0], adjb.at[an1], sem_a.at[an1]).wait()
    pltpu.sync_copy(acc, out_hbm.at[pl.ds(pl.multiple_of(row_base, 8), R)])


@functools.lru_cache(maxsize=None)
def _get_sc_hop():
    return functools.partial(
        pl.kernel,
        out_type=jax.ShapeDtypeStruct((NP, NCLASS), jnp.float32),
        mesh=plsc.VectorSubcoreMesh(core_axis_name="c", subcore_axis_name="s",
                                    num_cores=2, num_subcores=16),
        scratch_types=[
            pltpu.VMEM((R, 64), jnp.float32),
            pltpu.VMEM((3, 3, C), jnp.int32),
            pltpu.VMEM((2, C, 64), jnp.float32),
            pltpu.VMEM((NW * 16,), jnp.int32),
            pltpu.SemaphoreType.DMA((3,)),
            pltpu.SemaphoreType.DMA((2,)),
        ],
        compiler_params=pltpu.CompilerParams(use_tc_tiling_on_sc=False,
                                             needs_layout_passes=False),
    )(_sc_hop_body)


def _combine_body(w_ref, *refs):
    fp_refs = refs[:DEGREE]
    out_ref = refs[DEGREE]
    w = w_ref[...]
    accs = []
    for h2 in range(2):
        sl = slice(h2 * NCLASS, (h2 + 1) * NCLASS)
        acc = w[:, h2 * TOPK:h2 * TOPK + 1] * fp_refs[0][...][:, sl]
        for h in range(1, DEGREE):
            acc = acc + (w[:, h2 * TOPK + h:h2 * TOPK + h + 1]
                         * fp_refs[h][...][:, sl])
        m = jnp.max(acc, axis=1, keepdims=True)
        s = acc - m
        accs.append(s - jnp.log(jnp.sum(jnp.exp(s), axis=1, keepdims=True)))
    out_ref[...] = jnp.concatenate(accs, axis=1)


def _combine_call(weight2, fps2):
    return pl.pallas_call(
        _combine_body,
        grid=(GRID,),
        in_specs=[pl.BlockSpec((BLK // 2, 2 * TOPK), lambda i: (i, 0))]
        + [pl.BlockSpec((BLK // 2, 2 * NCLASS), lambda i: (i, 0))] * DEGREE,
        out_specs=pl.BlockSpec((BLK // 2, 2 * NCLASS), lambda i: (i, 0)),
        out_shape=jax.ShapeDtypeStruct((NP // 2, 2 * NCLASS), jnp.float32),
    )(weight2, *fps2)


def kernel(f_prediction, g0_prediction, adj_row, adj_col, adj_val,
           W1, b1, W2, b2, W3, b3):
    f_pad = jnp.pad(f_prediction, ((0, NP - N), (0, 0)))
    g0_pad = jnp.pad(g0_prediction, ((0, NP - N), (0, 0)))
    W3p = jnp.pad(W3, ((0, 0), (0, TOPK - DEGREE)))
    b1r = b1.reshape(1, -1)
    b2r = b2.reshape(1, -1)
    b3r = jnp.pad(b3, (0, TOPK - DEGREE)).reshape(1, -1)

    row32 = adj_row.astype(jnp.int32)
    bounds = jnp.searchsorted(row32, jnp.arange(NW + 1, dtype=jnp.int32) * R,
                              side="left").astype(jnp.int32)
    bounds_arr = jnp.zeros((NW, 16), jnp.int32)
    bounds_arr = bounds_arr.at[:, 0].set(bounds[:NW])
    bounds_arr = bounds_arr.at[:, 1].set(bounds[1:]).reshape(NW * 16)
    pad_e = NCH * C - E
    colp = jnp.pad(adj_col.astype(jnp.int32), (0, pad_e)).reshape(NCH, 1, C)
    valp = jnp.pad(lax.bitcast_convert_type(adj_val, jnp.int32),
                   (0, pad_e)).reshape(NCH, 1, C)
    rowp = jnp.pad(row32, (0, pad_e)).reshape(NCH, 1, C)
    adj_packed = jnp.concatenate([colp, valp, rowp], axis=1)

    sc_hop = _get_sc_hop()
    fps = [f_pad]
    cur = f_pad
    weight_pad = None
    for h in range(DEGREE - 1):
        cur = sc_hop(cur, adj_packed, bounds_arr)
        fps.append(cur)
        if h == 0:
            weight_pad = _mlp_call(g0_pad, W1, b1r, W2, b2r, W3p, b3r)

    fps2 = [fp.reshape(NP // 2, 2 * NCLASS) for fp in fps]
    weight2 = weight_pad.reshape(NP // 2, 2 * TOPK)
    logp_pad = _combine_call(weight2, fps2).reshape(NP, NCLASS)
    return logp_pad[:N], weight_pad[:N, :DEGREE]


# speedup vs baseline: 16.8386x; 1.3008x over previous
import functools

import jax
import jax.numpy as jnp
from jax import lax
from jax.experimental import pallas as pl
from jax.experimental.pallas import tpu as pltpu
from jax.experimental.pallas import tpu_sc as plsc

N = 50000
E = 800000
NCLASS = 64
TOPK = 16
DEGREE = 9

NW = 32
R = 1568
NP = NW * R
BLK = 512
GRID = NP // BLK
C = 128


def _mlp_body(g0_ref, w1_ref, b1_ref, w2_ref, b2_ref, w3_ref, b3_ref, out_ref):
    x = g0_ref[...]
    m = jnp.max(x, axis=1, keepdims=True)
    e = jnp.exp(x - m)
    p = e / jnp.sum(e, axis=1, keepdims=True)
    iota = lax.broadcasted_iota(jnp.int32, p.shape, 1)
    cur = p + iota.astype(jnp.float32) * (2.0 ** -20)
    cols = []
    for _ in range(TOPK):
        mx = jnp.max(cur, axis=1, keepdims=True)
        cols.append(mx)
        cur = jnp.where(cur == mx, -jnp.inf, cur)
    ranked = jnp.concatenate(cols, axis=1)
    hp = jax.lax.Precision.HIGHEST
    h = jnp.tanh(jnp.dot(ranked, w1_ref[...], precision=hp,
                         preferred_element_type=jnp.float32) + b1_ref[...])
    h = jnp.tanh(jnp.dot(h, w2_ref[...], precision=hp,
                         preferred_element_type=jnp.float32) + b2_ref[...])
    out_ref[...] = jnp.dot(h, w3_ref[...], precision=hp,
                           preferred_element_type=jnp.float32) + b3_ref[...]


def _mlp_call(g0_pad, W1, b1, W2, b2, W3p, b3p):
    full = lambda s: pl.BlockSpec(s, lambda i: (0, 0))
    return pl.pallas_call(
        _mlp_body,
        grid=(GRID,),
        in_specs=[
            pl.BlockSpec((BLK, NCLASS), lambda i: (i, 0)),
            full(W1.shape), full(b1.shape), full(W2.shape),
            full(b2.shape), full(W3p.shape), full(b3p.shape),
        ],
        out_specs=pl.BlockSpec((BLK, TOPK), lambda i: (i, 0)),
        out_shape=jax.ShapeDtypeStruct((NP, TOPK), jnp.float32),
    )(g0_pad, W1, b1, W2, b2, W3p, b3p)


def _sc_bounds_body(row_hbm, out_hbm, buf, stage):
    cid = lax.axis_index("c")
    sid = lax.axis_index("s")
    wid = sid * 2 + cid
    lanes = lax.iota(jnp.int32, 16)
    nblk = E // 16

    def search(q):
        def step(_, lh):
            lo, hi = lh
            mid = lax.div(lo + hi, 2)
            pltpu.sync_copy(
                row_hbm.at[pl.ds(pl.multiple_of(mid * 16, 8), 16)], buf)
            v = buf[pl.ds(0, 16)][0]
            pred = v < q
            return (jnp.where(pred, mid + 1, lo), jnp.where(pred, hi, mid))
        lo, _ = lax.fori_loop(0, 16, step, (jnp.int32(0), jnp.int32(nblk)))
        lm1 = jnp.maximum(lo, 1) - 1
        pltpu.sync_copy(
            row_hbm.at[pl.ds(pl.multiple_of(lm1 * 16, 8), 16)], buf)
        cnt = plsc.all_reduce_population_count(buf[pl.ds(0, 16)] < q)[0]
        return jnp.where(lo == 0, 0, lm1 * 16 + cnt)

    start = search(wid * R)
    end = search((wid + 1) * R)
    stage[...] = jnp.where(lanes == 0, start,
                           jnp.where(lanes == 1, end, 0)).astype(jnp.int32)
    pltpu.sync_copy(stage, out_hbm.at[wid])


@functools.lru_cache(maxsize=None)
def _get_sc_bounds():
    return functools.partial(
        pl.kernel,
        out_type=jax.ShapeDtypeStruct((NW, 16), jnp.int32),
        mesh=plsc.VectorSubcoreMesh(core_axis_name="c", subcore_axis_name="s",
                                    num_cores=2, num_subcores=16),
        scratch_types=[
            pltpu.VMEM((16,), jnp.int32),
            pltpu.VMEM((16,), jnp.int32),
        ],
        compiler_params=pltpu.CompilerParams(use_tc_tiling_on_sc=False,
                                             needs_layout_passes=False),
    )(_sc_bounds_body)


def _sc_hop_body(fp_hbm, col_hbm, val_hbm, row_hbm, bounds_hbm, out_hbm,
                 acc, colb, valb, rowb, rowsb, bv, sem_a, sem_g):
    cid = lax.axis_index("c")
    sid = lax.axis_index("s")
    wid = sid * 2 + cid
    pltpu.sync_copy(bounds_hbm, bv)
    bpair = bv[pl.ds(wid * 16, 16)]
    start = bpair[0]
    end = bpair[1]
    row_base = wid * R

    k0 = lax.div(start, C)
    nch = lax.div(end + (C - 1), C) - k0
    lanes = lax.iota(jnp.int32, 16)
    splat_idx = [jnp.full((16,), i, jnp.int32) for i in range(16)]
    off_j = [j * 16 + lanes for j in range(4)]

    def adj_src(k):
        return pl.ds(pl.multiple_of(jnp.minimum(k * C, E - C), 8), C)

    def adj_issue(k, a):
        pltpu.async_copy(col_hbm.at[adj_src(k)], colb.at[a], sem_a.at[a])
        pltpu.async_copy(val_hbm.at[adj_src(k)], valb.at[a], sem_a.at[a])
        pltpu.async_copy(row_hbm.at[adj_src(k)], rowb.at[a], sem_a.at[a])

    def adj_wait(k, a):
        pltpu.make_async_copy(col_hbm.at[adj_src(k)], colb.at[a],
                              sem_a.at[a]).wait()
        pltpu.make_async_copy(val_hbm.at[adj_src(k)], valb.at[a],
                              sem_a.at[a]).wait()
        pltpu.make_async_copy(row_hbm.at[adj_src(k)], rowb.at[a],
                              sem_a.at[a]).wait()

    adj_issue(k0, 0)
    z16 = jnp.zeros((16,), jnp.float32)

    def zero(i, _):
        for j in range(4):
            acc[i, pl.ds(j * 16, 16)] = z16
        return 0
    lax.fori_loop(0, R, zero, 0)
    adj_wait(k0, 0)
    pltpu.async_copy(fp_hbm.at[colb.at[0]], rowsb.at[0], sem_g.at[0])
    adj_issue(k0 + 1, 1)

    def chunk(t, _):
        a = lax.rem(t, 3)
        b = lax.rem(t, 2)
        a1 = lax.rem(t + 1, 3)
        b1 = lax.rem(t + 1, 2)
        a2 = lax.rem(t + 2, 3)
        adj_wait(k0 + t + 1, a1)
        pltpu.async_copy(fp_hbm.at[colb.at[a1]], rowsb.at[b1], sem_g.at[b1])
        adj_issue(k0 + t + 2, a2)
        pltpu.make_async_copy(fp_hbm.at[colb.at[a]], rowsb.at[b],
                              sem_g.at[b]).wait()
        off = (k0 + t) * C
        for g in range(C // 16):
            eidx = off + g * 16 + lanes
            ok = (eidx >= start) & (eidx < end)
            rv = jnp.clip(rowb[a, pl.ds(g * 16, 16)] - row_base, 0, R - 1)
            vv = jnp.where(ok, valb[a, pl.ds(g * 16, 16)], 0.0)
            for q in range(4):
                e4 = [q * 4 + u for u in range(4)]
                idx = [rv.at[splat_idx[i]].get(mode="promise_in_bounds")
                       for i in e4]
                vs = [vv.at[splat_idx[i]].get(mode="promise_in_bounds")
                      for i in e4]
                xs = [[vs[u] * rowsb[b, g * 16 + e4[u], pl.ds(j * 16, 16)]
                       for j in range(4)] for u in range(4)]
                for u in range(4):
                    for j in range(4):
                        plsc.addupdate_scatter(acc, [idx[u], off_j[j]],
                                               xs[u][j])
        return 0
    lax.fori_loop(0, nch, chunk, 0)

    bn = lax.rem(nch, 2)
    an = lax.rem(nch, 3)
    an1 = lax.rem(nch + 1, 3)
    pltpu.make_async_copy(fp_hbm.at[colb.at[an]], rowsb.at[bn],
                          sem_g.at[bn]).wait()
    adj_wait(k0 + nch + 1, an1)
    pltpu.sync_copy(acc, out_hbm.at[pl.ds(pl.multiple_of(row_base, 8), R)])


@functools.lru_cache(maxsize=None)
def _get_sc_hop():
    return functools.partial(
        pl.kernel,
        out_type=jax.ShapeDtypeStruct((NP, NCLASS), jnp.float32),
        mesh=plsc.VectorSubcoreMesh(core_axis_name="c", subcore_axis_name="s",
                                    num_cores=2, num_subcores=16),
        scratch_types=[
            pltpu.VMEM((R, 64), jnp.float32),
            pltpu.VMEM((3, C), jnp.int32),
            pltpu.VMEM((3, C), jnp.float32),
            pltpu.VMEM((3, C), jnp.int32),
            pltpu.VMEM((2, C, 64), jnp.float32),
            pltpu.VMEM((NW * 16,), jnp.int32),
            pltpu.SemaphoreType.DMA((3,)),
            pltpu.SemaphoreType.DMA((2,)),
        ],
        compiler_params=pltpu.CompilerParams(use_tc_tiling_on_sc=False,
                                             needs_layout_passes=False),
    )(_sc_hop_body)


def _combine_body(w_ref, *refs):
    fp_refs = refs[:DEGREE]
    out_ref = refs[DEGREE]
    w = w_ref[...]
    accs = []
    for h2 in range(2):
        sl = slice(h2 * NCLASS, (h2 + 1) * NCLASS)
        acc = w[:, h2 * TOPK:h2 * TOPK + 1] * fp_refs[0][...][:, sl]
        for h in range(1, DEGREE):
            acc = acc + (w[:, h2 * TOPK + h:h2 * TOPK + h + 1]
                         * fp_refs[h][...][:, sl])
        m = jnp.max(acc, axis=1, keepdims=True)
        s = acc - m
        accs.append(s - jnp.log(jnp.sum(jnp.exp(s), axis=1, keepdims=True)))
    out_ref[...] = jnp.concatenate(accs, axis=1)


def _combine_call(weight2, fps2):
    return pl.pallas_call(
        _combine_body,
        grid=(GRID,),
        in_specs=[pl.BlockSpec((BLK // 2, 2 * TOPK), lambda i: (i, 0))]
        + [pl.BlockSpec((BLK // 2, 2 * NCLASS), lambda i: (i, 0))] * DEGREE,
        out_specs=pl.BlockSpec((BLK // 2, 2 * NCLASS), lambda i: (i, 0)),
        out_shape=jax.ShapeDtypeStruct((NP // 2, 2 * NCLASS), jnp.float32),
    )(weight2, *fps2)


def kernel(f_prediction, g0_prediction, adj_row, adj_col, adj_val,
           W1, b1, W2, b2, W3, b3):
    f_pad = jnp.pad(f_prediction, ((0, NP - N), (0, 0)))
    g0_pad = jnp.pad(g0_prediction, ((0, NP - N), (0, 0)))
    W3p = jnp.pad(W3, ((0, 0), (0, TOPK - DEGREE)))
    b1r = b1.reshape(1, -1)
    b2r = b2.reshape(1, -1)
    b3r = jnp.pad(b3, (0, TOPK - DEGREE)).reshape(1, -1)

    row32 = adj_row.astype(jnp.int32)
    col32 = adj_col.astype(jnp.int32)
    bounds_arr = _get_sc_bounds()(row32).reshape(NW * 16)

    sc_hop = _get_sc_hop()
    fps = [f_pad]
    cur = f_pad
    weight_pad = None
    for h in range(DEGREE - 1):
        cur = sc_hop(cur, col32, adj_val, row32, bounds_arr)
        fps.append(cur)
        if h == 0:
            weight_pad = _mlp_call(g0_pad, W1, b1r, W2, b2r, W3p, b3r)

    fps2 = [fp.reshape(NP // 2, 2 * NCLASS) for fp in fps]
    weight2 = weight_pad.reshape(NP // 2, 2 * TOPK)
    logp_pad = _combine_call(weight2, fps2).reshape(NP, NCLASS)
    return logp_pad[:N], weight_pad[:N, :DEGREE]


# speedup vs baseline: 17.2777x; 1.0261x over previous
import functools

import jax
import jax.numpy as jnp
from jax import lax
from jax.experimental import pallas as pl
from jax.experimental.pallas import tpu as pltpu
from jax.experimental.pallas import tpu_sc as plsc

N = 50000
E = 800000
NCLASS = 64
TOPK = 16
DEGREE = 9

NW = 32
R = 1568
NP = NW * R
BLK = 512
GRID = NP // BLK
C = 128


def _mlp_body(g0_ref, w1_ref, b1_ref, w2_ref, b2_ref, w3_ref, b3_ref, out_ref):
    x = g0_ref[...]
    m = jnp.max(x, axis=1, keepdims=True)
    e = jnp.exp(x - m)
    p = e / jnp.sum(e, axis=1, keepdims=True)
    iota = lax.broadcasted_iota(jnp.int32, p.shape, 1)
    cur = p + iota.astype(jnp.float32) * (2.0 ** -21)
    cols = []
    for _ in range(TOPK):
        mx = jnp.max(cur, axis=1, keepdims=True)
        cols.append(mx)
        cur = jnp.where(cur == mx, -jnp.inf, cur)
    ranked = jnp.concatenate(cols, axis=1)
    h = jnp.tanh(jnp.dot(ranked, w1_ref[...],
                         preferred_element_type=jnp.float32) + b1_ref[...])
    h = jnp.tanh(jnp.dot(h, w2_ref[...],
                         preferred_element_type=jnp.float32) + b2_ref[...])
    out_ref[...] = jnp.dot(h, w3_ref[...],
                           preferred_element_type=jnp.float32) + b3_ref[...]


def _mlp_call(g0_pad, W1, b1, W2, b2, W3p, b3p):
    full = lambda s: pl.BlockSpec(s, lambda i: (0, 0))
    return pl.pallas_call(
        _mlp_body,
        grid=(GRID,),
        in_specs=[
            pl.BlockSpec((BLK, NCLASS), lambda i: (i, 0)),
            full(W1.shape), full(b1.shape), full(W2.shape),
            full(b2.shape), full(W3p.shape), full(b3p.shape),
        ],
        out_specs=pl.BlockSpec((BLK, TOPK), lambda i: (i, 0)),
        out_shape=jax.ShapeDtypeStruct((NP, TOPK), jnp.float32),
    )(g0_pad, W1, b1, W2, b2, W3p, b3p)


def _sc_bounds_body(row_hbm, out_hbm, buf, stage):
    cid = lax.axis_index("c")
    sid = lax.axis_index("s")
    wid = sid * 2 + cid
    lanes = lax.iota(jnp.int32, 16)
    nblk = E // 16

    def search(q):
        def step(_, lh):
            lo, hi = lh
            mid = lax.div(lo + hi, 2)
            pltpu.sync_copy(
                row_hbm.at[pl.ds(pl.multiple_of(mid * 16, 8), 16)], buf)
            v = buf[pl.ds(0, 16)][0]
            pred = v < q
            return (jnp.where(pred, mid + 1, lo), jnp.where(pred, hi, mid))
        lo, _ = lax.fori_loop(0, 16, step, (jnp.int32(0), jnp.int32(nblk)))
        lm1 = jnp.maximum(lo, 1) - 1
        pltpu.sync_copy(
            row_hbm.at[pl.ds(pl.multiple_of(lm1 * 16, 8), 16)], buf)
        cnt = plsc.all_reduce_population_count(buf[pl.ds(0, 16)] < q)[0]
        return jnp.where(lo == 0, 0, lm1 * 16 + cnt)

    start = search(wid * R)
    end = search((wid + 1) * R)
    stage[...] = jnp.where(lanes == 0, start,
                           jnp.where(lanes == 1, end, 0)).astype(jnp.int32)
    pltpu.sync_copy(stage, out_hbm.at[wid])


@functools.lru_cache(maxsize=None)
def _get_sc_bounds():
    return functools.partial(
        pl.kernel,
        out_type=jax.ShapeDtypeStruct((NW, 16), jnp.int32),
        mesh=plsc.VectorSubcoreMesh(core_axis_name="c", subcore_axis_name="s",
                                    num_cores=2, num_subcores=16),
        scratch_types=[
            pltpu.VMEM((16,), jnp.int32),
            pltpu.VMEM((16,), jnp.int32),
        ],
        compiler_params=pltpu.CompilerParams(use_tc_tiling_on_sc=False,
                                             needs_layout_passes=False),
    )(_sc_bounds_body)


def _sc_hop_body(fp_hbm, col_hbm, val_hbm, row_hbm, bounds_hbm, out_hbm,
                 acc, colb, valb, rowb, rowsb, bv, sem_a, sem_g):
    cid = lax.axis_index("c")
    sid = lax.axis_index("s")
    wid = sid * 2 + cid
    pltpu.sync_copy(bounds_hbm, bv)
    bpair = bv[pl.ds(wid * 16, 16)]
    start = bpair[0]
    end = bpair[1]
    row_base = wid * R

    k0 = lax.div(start, C)
    nch = lax.div(end + (C - 1), C) - k0
    lanes = lax.iota(jnp.int32, 16)
    splat_idx = [jnp.full((16,), i, jnp.int32) for i in range(16)]
    off_j = [j * 16 + lanes for j in range(4)]

    def adj_src(k):
        return pl.ds(pl.multiple_of(jnp.minimum(k * C, E - C), 8), C)

    def adj_issue(k, a):
        pltpu.async_copy(col_hbm.at[adj_src(k)], colb.at[a], sem_a.at[a])
        pltpu.async_copy(val_hbm.at[adj_src(k)], valb.at[a], sem_a.at[a])
        pltpu.async_copy(row_hbm.at[adj_src(k)], rowb.at[a], sem_a.at[a])

    def adj_wait(k, a):
        pltpu.make_async_copy(col_hbm.at[adj_src(k)], colb.at[a],
                              sem_a.at[a]).wait()
        pltpu.make_async_copy(val_hbm.at[adj_src(k)], valb.at[a],
                              sem_a.at[a]).wait()
        pltpu.make_async_copy(row_hbm.at[adj_src(k)], rowb.at[a],
                              sem_a.at[a]).wait()

    adj_issue(k0, 0)
    adj_issue(k0 + 1, 1)
    z16 = jnp.zeros((16,), jnp.float32)

    def zero(i, _):
        for j in range(4):
            acc[i, pl.ds(j * 16, 16)] = z16
        return 0
    lax.fori_loop(0, R, zero, 0)
    adj_wait(k0, 0)
    pltpu.async_copy(fp_hbm.at[colb.at[0]], rowsb.at[0], sem_g.at[0])
    adj_wait(k0 + 1, 1)
    pltpu.async_copy(fp_hbm.at[colb.at[1]], rowsb.at[1], sem_g.at[1])
    adj_issue(k0 + 2, 2)

    def chunk(t, _):
        a = lax.rem(t, 4)
        b = lax.rem(t, 3)
        a2 = lax.rem(t + 2, 4)
        b2 = lax.rem(t + 2, 3)
        a3 = lax.rem(t + 3, 4)
        adj_wait(k0 + t + 2, a2)
        pltpu.async_copy(fp_hbm.at[colb.at[a2]], rowsb.at[b2], sem_g.at[b2])
        adj_issue(k0 + t + 3, a3)
        pltpu.make_async_copy(fp_hbm.at[colb.at[a]], rowsb.at[b],
                              sem_g.at[b]).wait()
        off = (k0 + t) * C
        for g in range(C // 16):
            eidx = off + g * 16 + lanes
            ok = (eidx >= start) & (eidx < end)
            g16 = g * 16
            rv = jnp.clip(rowb[a, pl.ds(g16, 16)] - row_base, 0, R - 1)
            vv = jnp.where(ok, valb[a, pl.ds(g16, 16)], 0.0)
            for q in range(4):
                e4 = [q * 4 + u for u in range(4)]
                idx = [rv.at[splat_idx[i]].get(mode="promise_in_bounds")
                       for i in e4]
                vs = [vv.at[splat_idx[i]].get(mode="promise_in_bounds")
                      for i in e4]
                xs = [[vs[u] * rowsb[b, g16 + e4[u], pl.ds(j * 16, 16)]
                       for j in range(4)] for u in range(4)]
                for u in range(4):
                    for j in range(4):
                        plsc.addupdate_scatter(acc, [idx[u], off_j[j]],
                                               xs[u][j])
        return 0
    lax.fori_loop(0, nch, chunk, 0)

    for d in range(2):
        bn = lax.rem(nch + d, 3)
        an = lax.rem(nch + d, 4)
        pltpu.make_async_copy(fp_hbm.at[colb.at[an]], rowsb.at[bn],
                              sem_g.at[bn]).wait()
    adj_wait(k0 + nch + 2, lax.rem(nch + 2, 4))
    pltpu.sync_copy(acc, out_hbm.at[pl.ds(pl.multiple_of(row_base, 8), R)])


@functools.lru_cache(maxsize=None)
def _get_sc_hop():
    return functools.partial(
        pl.kernel,
        out_type=jax.ShapeDtypeStruct((NP, NCLASS), jnp.float32),
        mesh=plsc.VectorSubcoreMesh(core_axis_name="c", subcore_axis_name="s",
                                    num_cores=2, num_subcores=16),
        scratch_types=[
            pltpu.VMEM((R, 64), jnp.float32),
            pltpu.VMEM((4, C), jnp.int32),
            pltpu.VMEM((4, C), jnp.float32),
            pltpu.VMEM((4, C), jnp.int32),
            pltpu.VMEM((3, C, 64), jnp.float32),
            pltpu.VMEM((NW * 16,), jnp.int32),
            pltpu.SemaphoreType.DMA((4,)),
            pltpu.SemaphoreType.DMA((3,)),
        ],
        compiler_params=pltpu.CompilerParams(use_tc_tiling_on_sc=False,
                                             needs_layout_passes=False),
    )(_sc_hop_body)


def _combine_body(w_ref, *refs):
    fp_refs = refs[:DEGREE]
    out_ref = refs[DEGREE]
    w = w_ref[...]
    accs = []
    for h2 in range(2):
        sl = slice(h2 * NCLASS, (h2 + 1) * NCLASS)
        acc = w[:, h2 * TOPK:h2 * TOPK + 1] * fp_refs[0][...][:, sl]
        for h in range(1, DEGREE):
            acc = acc + (w[:, h2 * TOPK + h:h2 * TOPK + h + 1]
                         * fp_refs[h][...][:, sl])
        m = jnp.max(acc, axis=1, keepdims=True)
        s = acc - m
        accs.append(s - jnp.log(jnp.sum(jnp.exp(s), axis=1, keepdims=True)))
    out_ref[...] = jnp.concatenate(accs, axis=1)


def _combine_call(weight2, fps2):
    return pl.pallas_call(
        _combine_body,
        grid=(GRID,),
        in_specs=[pl.BlockSpec((BLK // 2, 2 * TOPK), lambda i: (i, 0))]
        + [pl.BlockSpec((BLK // 2, 2 * NCLASS), lambda i: (i, 0))] * DEGREE,
        out_specs=pl.BlockSpec((BLK // 2, 2 * NCLASS), lambda i: (i, 0)),
        out_shape=jax.ShapeDtypeStruct((N // 2, 2 * NCLASS), jnp.float32),
    )(weight2, *fps2)


def kernel(f_prediction, g0_prediction, adj_row, adj_col, adj_val,
           W1, b1, W2, b2, W3, b3):
    f_pad = jnp.pad(f_prediction, ((0, NP - N), (0, 0)))
    g0_pad = jnp.pad(g0_prediction, ((0, NP - N), (0, 0)))
    W3p = jnp.pad(W3, ((0, 0), (0, TOPK - DEGREE)))
    b1r = b1.reshape(1, -1)
    b2r = b2.reshape(1, -1)
    b3r = jnp.pad(b3, (0, TOPK - DEGREE)).reshape(1, -1)

    row32 = adj_row.astype(jnp.int32)
    col32 = adj_col.astype(jnp.int32)
    bounds_arr = _get_sc_bounds()(row32).reshape(NW * 16)

    sc_hop = _get_sc_hop()
    fps = [f_pad]
    cur = f_pad
    weight_pad = None
    for h in range(DEGREE - 1):
        cur = sc_hop(cur, col32, adj_val, row32, bounds_arr)
        fps.append(cur)
        if h == 0:
            weight_pad = _mlp_call(g0_pad, W1, b1r, W2, b2r, W3p, b3r)

    fps2 = [fp.reshape(NP // 2, 2 * NCLASS) for fp in fps]
    weight2 = weight_pad.reshape(NP // 2, 2 * TOPK)
    logp = _combine_call(weight2, fps2).reshape(N, NCLASS)
    return logp, weight_pad[:N, :DEGREE]


# speedup vs baseline: 17.2875x; 1.0006x over previous
import functools

import jax
import jax.numpy as jnp
from jax import lax
from jax.experimental import pallas as pl
from jax.experimental.pallas import tpu as pltpu
from jax.experimental.pallas import tpu_sc as plsc

N = 50000
E = 800000
NCLASS = 64
TOPK = 16
DEGREE = 9

NW = 32
R = 1568
NP = NW * R
BLK = 512
GRID = NP // BLK
C = 128


def _mlp_body(g0_ref, w1_ref, b1_ref, w2_ref, b2_ref, w3_ref, b3_ref,
              out_ref, out9_ref):
    x = g0_ref[...]
    m = jnp.max(x, axis=1, keepdims=True)
    e = jnp.exp(x - m)
    p = e / jnp.sum(e, axis=1, keepdims=True)
    iota = lax.broadcasted_iota(jnp.int32, p.shape, 1)
    cur = p + iota.astype(jnp.float32) * (2.0 ** -21)
    cols = []
    for _ in range(TOPK):
        mx = jnp.max(cur, axis=1, keepdims=True)
        cols.append(mx)
        cur = jnp.where(cur == mx, -jnp.inf, cur)
    ranked = jnp.concatenate(cols, axis=1)
    h = jnp.tanh(jnp.dot(ranked, w1_ref[...],
                         preferred_element_type=jnp.float32) + b1_ref[...])
    h = jnp.tanh(jnp.dot(h, w2_ref[...],
                         preferred_element_type=jnp.float32) + b2_ref[...])
    w = jnp.dot(h, w3_ref[...],
                preferred_element_type=jnp.float32) + b3_ref[...]
    out_ref[...] = w
    out9_ref[...] = w[:, :DEGREE]


def _mlp_call(g0_pad, W1, b1, W2, b2, W3p, b3p):
    full = lambda s: pl.BlockSpec(s, lambda i: (0, 0))
    return pl.pallas_call(
        _mlp_body,
        grid=(GRID,),
        in_specs=[
            pl.BlockSpec((BLK, NCLASS), lambda i: (i, 0)),
            full(W1.shape), full(b1.shape), full(W2.shape),
            full(b2.shape), full(W3p.shape), full(b3p.shape),
        ],
        out_specs=[pl.BlockSpec((BLK, TOPK), lambda i: (i, 0)),
                   pl.BlockSpec((BLK, DEGREE), lambda i: (i, 0))],
        out_shape=[jax.ShapeDtypeStruct((NP, TOPK), jnp.float32),
                   jax.ShapeDtypeStruct((N, DEGREE), jnp.float32)],
    )(g0_pad, W1, b1, W2, b2, W3p, b3p)


def _sc_bounds_body(row_hbm, out_hbm, buf, stage):
    cid = lax.axis_index("c")
    sid = lax.axis_index("s")
    wid = sid * 2 + cid
    lanes = lax.iota(jnp.int32, 16)
    nblk = E // 16

    def search(q):
        def step(_, lh):
            lo, hi = lh
            mid = lax.div(lo + hi, 2)
            pltpu.sync_copy(
                row_hbm.at[pl.ds(pl.multiple_of(mid * 16, 8), 16)], buf)
            v = buf[pl.ds(0, 16)][0]
            pred = v < q
            return (jnp.where(pred, mid + 1, lo), jnp.where(pred, hi, mid))
        lo, _ = lax.fori_loop(0, 16, step, (jnp.int32(0), jnp.int32(nblk)))
        lm1 = jnp.maximum(lo, 1) - 1
        pltpu.sync_copy(
            row_hbm.at[pl.ds(pl.multiple_of(lm1 * 16, 8), 16)], buf)
        cnt = plsc.all_reduce_population_count(buf[pl.ds(0, 16)] < q)[0]
        return jnp.where(lo == 0, 0, lm1 * 16 + cnt)

    start = search(wid * R)
    end = search((wid + 1) * R)
    stage[...] = jnp.where(lanes == 0, start,
                           jnp.where(lanes == 1, end, 0)).astype(jnp.int32)
    pltpu.sync_copy(stage, out_hbm.at[wid])


@functools.lru_cache(maxsize=None)
def _get_sc_bounds():
    return functools.partial(
        pl.kernel,
        out_type=jax.ShapeDtypeStruct((NW, 16), jnp.int32),
        mesh=plsc.VectorSubcoreMesh(core_axis_name="c", subcore_axis_name="s",
                                    num_cores=2, num_subcores=16),
        scratch_types=[
            pltpu.VMEM((16,), jnp.int32),
            pltpu.VMEM((16,), jnp.int32),
        ],
        compiler_params=pltpu.CompilerParams(use_tc_tiling_on_sc=False,
                                             needs_layout_passes=False),
    )(_sc_bounds_body)


def _sc_hop_body(fp_hbm, col_hbm, val_hbm, row_hbm, bounds_hbm, out_hbm,
                 acc, colb, valb, rowb, rowsb, bv, sem_a, sem_g):
    cid = lax.axis_index("c")
    sid = lax.axis_index("s")
    wid = sid * 2 + cid
    pltpu.sync_copy(bounds_hbm, bv)
    bpair = bv[pl.ds(wid * 16, 16)]
    start = bpair[0]
    end = bpair[1]
    row_base = wid * R

    k0 = lax.div(start, C)
    nch = lax.div(end + (C - 1), C) - k0
    lanes = lax.iota(jnp.int32, 16)
    splat_idx = [jnp.full((16,), i, jnp.int32) for i in range(16)]
    off_j = [j * 16 + lanes for j in range(4)]

    def adj_src(k):
        return pl.ds(pl.multiple_of(jnp.minimum(k * C, E - C), 8), C)

    def adj_issue(k, a):
        pltpu.async_copy(col_hbm.at[adj_src(k)], colb.at[a], sem_a.at[a])
        pltpu.async_copy(val_hbm.at[adj_src(k)], valb.at[a], sem_a.at[a])
        pltpu.async_copy(row_hbm.at[adj_src(k)], rowb.at[a], sem_a.at[a])

    def adj_wait(k, a):
        pltpu.make_async_copy(col_hbm.at[adj_src(k)], colb.at[a],
                              sem_a.at[a]).wait()
        pltpu.make_async_copy(val_hbm.at[adj_src(k)], valb.at[a],
                              sem_a.at[a]).wait()
        pltpu.make_async_copy(row_hbm.at[adj_src(k)], rowb.at[a],
                              sem_a.at[a]).wait()

    adj_issue(k0, 0)
    adj_issue(k0 + 1, 1)
    z16 = jnp.zeros((16,), jnp.float32)

    def zero(i, _):
        for j in range(4):
            acc[i, pl.ds(j * 16, 16)] = z16
        return 0
    lax.fori_loop(0, R, zero, 0)
    adj_wait(k0, 0)
    pltpu.async_copy(fp_hbm.at[colb.at[0]], rowsb.at[0], sem_g.at[0])
    adj_wait(k0 + 1, 1)
    pltpu.async_copy(fp_hbm.at[colb.at[1]], rowsb.at[1], sem_g.at[1])
    adj_issue(k0 + 2, 2)

    def chunk(t, _):
        a = lax.rem(t, 4)
        b = lax.rem(t, 3)
        a2 = lax.rem(t + 2, 4)
        b2 = lax.rem(t + 2, 3)
        a3 = lax.rem(t + 3, 4)
        adj_wait(k0 + t + 2, a2)
        pltpu.async_copy(fp_hbm.at[colb.at[a2]], rowsb.at[b2], sem_g.at[b2])
        adj_issue(k0 + t + 3, a3)
        pltpu.make_async_copy(fp_hbm.at[colb.at[a]], rowsb.at[b],
                              sem_g.at[b]).wait()
        off = (k0 + t) * C
        for g in range(C // 16):
            eidx = off + g * 16 + lanes
            ok = (eidx >= start) & (eidx < end)
            g16 = g * 16
            rv = jnp.clip(rowb[a, pl.ds(g16, 16)] - row_base, 0, R - 1)
            vv = jnp.where(ok, valb[a, pl.ds(g16, 16)], 0.0)
            for q in range(4):
                e4 = [q * 4 + u for u in range(4)]
                idx = [rv.at[splat_idx[i]].get(mode="promise_in_bounds")
                       for i in e4]
                vs = [vv.at[splat_idx[i]].get(mode="promise_in_bounds")
                      for i in e4]
                xs = [[vs[u] * rowsb[b, g16 + e4[u], pl.ds(j * 16, 16)]
                       for j in range(4)] for u in range(4)]
                for u in range(4):
                    for j in range(4):
                        plsc.addupdate_scatter(acc, [idx[u], off_j[j]],
                                               xs[u][j])
        return 0
    lax.fori_loop(0, nch, chunk, 0)

    for d in range(2):
        bn = lax.rem(nch + d, 3)
        an = lax.rem(nch + d, 4)
        pltpu.make_async_copy(fp_hbm.at[colb.at[an]], rowsb.at[bn],
                              sem_g.at[bn]).wait()
    adj_wait(k0 + nch + 2, lax.rem(nch + 2, 4))
    pltpu.sync_copy(acc, out_hbm.at[pl.ds(pl.multiple_of(row_base, 8), R)])


@functools.lru_cache(maxsize=None)
def _get_sc_hop():
    return functools.partial(
        pl.kernel,
        out_type=jax.ShapeDtypeStruct((NP, NCLASS), jnp.float32),
        mesh=plsc.VectorSubcoreMesh(core_axis_name="c", subcore_axis_name="s",
                                    num_cores=2, num_subcores=16),
        scratch_types=[
            pltpu.VMEM((R, 64), jnp.float32),
            pltpu.VMEM((4, C), jnp.int32),
            pltpu.VMEM((4, C), jnp.float32),
            pltpu.VMEM((4, C), jnp.int32),
            pltpu.VMEM((3, C, 64), jnp.float32),
            pltpu.VMEM((NW * 16,), jnp.int32),
            pltpu.SemaphoreType.DMA((4,)),
            pltpu.SemaphoreType.DMA((3,)),
        ],
        compiler_params=pltpu.CompilerParams(use_tc_tiling_on_sc=False,
                                             needs_layout_passes=False),
    )(_sc_hop_body)


def _combine_body(w_ref, *refs):
    fp_refs = refs[:DEGREE]
    out_ref = refs[DEGREE]
    w = w_ref[...]
    accs = []
    for h2 in range(2):
        sl = slice(h2 * NCLASS, (h2 + 1) * NCLASS)
        acc = w[:, h2 * TOPK:h2 * TOPK + 1] * fp_refs[0][...][:, sl]
        for h in range(1, DEGREE):
            acc = acc + (w[:, h2 * TOPK + h:h2 * TOPK + h + 1]
                         * fp_refs[h][...][:, sl])
        m = jnp.max(acc, axis=1, keepdims=True)
        s = acc - m
        accs.append(s - jnp.log(jnp.sum(jnp.exp(s), axis=1, keepdims=True)))
    out_ref[...] = jnp.concatenate(accs, axis=1)


def _combine_call(weight2, fps2):
    return pl.pallas_call(
        _combine_body,
        grid=(GRID,),
        in_specs=[pl.BlockSpec((BLK // 2, 2 * TOPK), lambda i: (i, 0))]
        + [pl.BlockSpec((BLK // 2, 2 * NCLASS), lambda i: (i, 0))] * DEGREE,
        out_specs=pl.BlockSpec((BLK // 2, 2 * NCLASS), lambda i: (i, 0)),
        out_shape=jax.ShapeDtypeStruct((N // 2, 2 * NCLASS), jnp.float32),
    )(weight2, *fps2)


def kernel(f_prediction, g0_prediction, adj_row, adj_col, adj_val,
           W1, b1, W2, b2, W3, b3):
    f_pad = jnp.pad(f_prediction, ((0, NP - N), (0, 0)))
    g0_pad = jnp.pad(g0_prediction, ((0, NP - N), (0, 0)))
    W3p = jnp.pad(W3, ((0, 0), (0, TOPK - DEGREE)))
    b1r = b1.reshape(1, -1)
    b2r = b2.reshape(1, -1)
    b3r = jnp.pad(b3, (0, TOPK - DEGREE)).reshape(1, -1)

    row32 = adj_row.astype(jnp.int32)
    col32 = adj_col.astype(jnp.int32)
    bounds_arr = _get_sc_bounds()(row32).reshape(NW * 16)

    sc_hop = _get_sc_hop()
    fps = [f_pad]
    cur = f_pad
    weight_pad = None
    for h in range(DEGREE - 1):
        cur = sc_hop(cur, col32, adj_val, row32, bounds_arr)
        fps.append(cur)
        if h == 0:
            weight_pad, weight9 = _mlp_call(g0_pad, W1, b1r, W2, b2r,
                                            W3p, b3r)

    fps2 = [fp.reshape(NP // 2, 2 * NCLASS) for fp in fps]
    weight2 = weight_pad.reshape(NP // 2, 2 * TOPK)
    logp = _combine_call(weight2, fps2).reshape(N, NCLASS)
    return logp, weight9
